# baseline (speedup 1.0000x reference)
import functools

import jax
import jax.numpy as jnp
from jax import lax
from jax.experimental import pallas as pl
from jax.experimental.pallas import tpu as pltpu
from jax.experimental.pallas import tpu_sc as plsc

N = 10000
E = 320000
D = 128
DA = 144
NC, NS = 2, 16
CB = 40
NB = 5
NCH = E // (NC * NS * CB)
HCH = NCH // 2
RPT = N // NS
BN = 2000


def _mm_body(a_ref, x0_ref, x1_ref, x2_ref, wl_ref, wr_ref, bb_ref,
             y0_ref, y1_ref, y2_ref, zb_ref):
    av = a_ref[0, 0]
    weights = (1.0, av, 1.0 - av)
    zb = bb_ref[0][None, :]
    for g, (x_ref, y_ref) in enumerate(
            ((x0_ref, y0_ref), (x1_ref, y1_ref), (x2_ref, y2_ref))):
        x = x_ref[...]
        y_ref[...] = jnp.dot(x, wl_ref[g], preferred_element_type=jnp.float32)
        zb = zb + weights[g] * jnp.dot(
            x, wr_ref[g], preferred_element_type=jnp.float32)
    zb_ref[...] = zb


def _matmuls(a2d, x0, x1, x2, wlts, wrts, bb):
    xspec = pl.BlockSpec((BN, D), lambda i: (i, 0))
    wspec = pl.BlockSpec((3, D, D), lambda i: (0, 0, 0))
    return pl.pallas_call(
        _mm_body,
        grid=(N // BN,),
        in_specs=[pl.BlockSpec(memory_space=pltpu.SMEM),
                  xspec, xspec, xspec, wspec, wspec,
                  pl.BlockSpec((1, D), lambda i: (0, 0))],
        out_specs=[xspec, xspec, xspec, xspec],
        out_shape=[jax.ShapeDtypeStruct((N, D), jnp.float32)] * 4,
    )(a2d, x0, x1, x2, wlts, wrts, bb)


def _sc_body(y0, y1, y2, e0, e1, e2, ones16, zf, zc,
             f0, f1, f2, c0, c1, c2,
             accf, accc, onesb, srcb, dstb, *bufs_and_sems):
    bufs = bufs_and_sems[:NB]
    gsems = bufs_and_sems[NB:2 * NB]
    fsems = bufs_and_sems[2 * NB:3 * NB]
    csems = bufs_and_sems[3 * NB:4 * NB]
    c = lax.axis_index("c")
    s = lax.axis_index("s")
    w = c * NS + s
    r0 = s * RPT
    ebase = w * NCH * CB

    pltpu.sync_copy(ones16, onesb)

    for y_hbm, e_hbm, fo, co in ((y0, e0, f0, c0), (y1, e1, f1, c1),
                                 (y2, e2, f2, c2)):
        pltpu.sync_copy(zf, accf.at[pl.ds(r0, RPT)])
        pltpu.sync_copy(zc, accc.at[pl.ds(r0, RPT)])
        plsc.subcore_barrier()

        for h in range(NCH // HCH):
            hbase = ebase + h * HCH * CB
            pltpu.sync_copy(e_hbm.at[0, pl.ds(hbase, HCH * CB)], srcb)
            pltpu.sync_copy(e_hbm.at[1, pl.ds(hbase, HCH * CB)], dstb)

            def gather(ch, j, y_hbm=y_hbm):
                return pltpu.make_async_copy(
                    y_hbm.at[srcb.at[pl.ds(ch * CB, CB)]], bufs[j], gsems[j])

            for j in range(NB):
                gather(j, j).start()

            def body(i, carry, gather=gather):
                base = NB * i
                scs = []
                for j in range(NB):
                    idx = dstb.at[pl.ds((base + j) * CB, CB)]
                    gather(base + j, j).wait()
                    scs.append(pltpu.async_copy(
                        bufs[j], accf.at[idx], fsems[j], add=True))
                    scs.append(pltpu.async_copy(
                        onesb, accc.at[idx], csems[j], add=True))
                for j in range(NB):
                    scs[2 * j].wait()
                    scs[2 * j + 1].wait()

                    @pl.when(i < HCH // NB - 1)
                    def _(j=j):
                        gather(base + NB + j, j).start()

                return carry

            lax.fori_loop(0, HCH // NB, body, 0)
        plsc.subcore_barrier()
        f8 = s * (RPT - 1)
        pltpu.sync_copy(accf.at[pl.ds(f8, RPT - 1)],
                        fo.at[c, pl.ds(f8, RPT - 1)])

        @pl.when(s == 0)
        def _(fo=fo):
            tail = NS * (RPT - 1)
            pltpu.sync_copy(accf.at[pl.ds(tail, N - tail)],
                            fo.at[c, pl.ds(tail, N - tail)])

        pltpu.sync_copy(accc.at[pl.ds(r0, RPT)], co.at[c, pl.ds(r0, RPT)])
        plsc.subcore_barrier()


@functools.cache
def _sc_call():
    return pl.kernel(
        _sc_body,
        out_type=[jax.ShapeDtypeStruct((NC, N, D), jnp.float32)] * 3
                 + [jax.ShapeDtypeStruct((NC, N, DA - D), jnp.float32)] * 3,
        mesh=plsc.VectorSubcoreMesh(
            core_axis_name="c", subcore_axis_name="s",
            num_cores=NC, num_subcores=NS),
        scratch_types=[
            pltpu.VMEM_SHARED((N, D), jnp.float32),
            pltpu.VMEM_SHARED((N, DA - D), jnp.float32),
            pltpu.VMEM((CB, DA - D), jnp.float32),
            pltpu.VMEM((HCH * CB,), jnp.int32),
            pltpu.VMEM((HCH * CB,), jnp.int32),
        ] + [pltpu.VMEM((CB, D), jnp.float32)] * NB
          + [pltpu.SemaphoreType.DMA] * (3 * NB),
        compiler_params=pltpu.CompilerParams(use_tc_tiling_on_sc=False),
    )


def _comb_body(a_ref, f0_ref, f1_ref, f2_ref, c0_ref, c1_ref, c2_ref,
               zb_ref, out_ref):
    av = a_ref[0, 0]
    weights = (1.0, av, 1.0 - av)
    tot = zb_ref[...]
    for g, (f_ref, c_ref) in enumerate(
            ((f0_ref, c0_ref), (f1_ref, c1_ref), (f2_ref, c2_ref))):
        sm = f_ref[0] + f_ref[1]
        cnt = c_ref[0, :, 0] + c_ref[1, :, 0]
        tot = tot + weights[g] * (sm / jnp.maximum(cnt, 1.0)[:, None])
    out_ref[...] = tot


def _combine(a2d, f0, f1, f2, c0, c1, c2, zb):
    fspec = pl.BlockSpec((NC, BN, D), lambda i: (0, i, 0))
    cspec = pl.BlockSpec((NC, BN, DA - D), lambda i: (0, i, 0))
    return pl.pallas_call(
        _comb_body,
        grid=(N // BN,),
        in_specs=[
            pl.BlockSpec(memory_space=pltpu.SMEM),
            fspec, fspec, fspec, cspec, cspec, cspec,
            pl.BlockSpec((BN, D), lambda i: (i, 0)),
        ],
        out_specs=pl.BlockSpec((BN, D), lambda i: (i, 0)),
        out_shape=jax.ShapeDtypeStruct((N, D), jnp.float32),
    )(a2d, f0, f1, f2, c0, c1, c2, zb)


def kernel(ui_x, ui_edge_index, s_x, s_edge_index, k_x, k_edge_index, a,
           W_l_ui, b_l_ui, W_r_ui, W_l_s, b_l_s, W_r_s, W_l_k, b_l_k, W_r_k):
    wlts = jnp.stack([W_l_ui.T, W_l_s.T, W_l_k.T])
    wrts = jnp.stack([W_r_ui.T, W_r_s.T, W_r_k.T])
    a2d = jnp.reshape(a, (1, 1))
    bb = (b_l_ui + a * b_l_s + (1.0 - a) * b_l_k).reshape(1, D)
    y0, y1, y2, zb = _matmuls(a2d, ui_x, s_x, k_x, wlts, wrts, bb)
    ones16 = jnp.concatenate(
        [jnp.ones((CB, 1), jnp.float32),
         jnp.zeros((CB, DA - D - 1), jnp.float32)], axis=1)
    zf = jnp.zeros((RPT, D), jnp.float32)
    zc = jnp.zeros((RPT, DA - D), jnp.float32)
    f0, f1, f2, c0, c1, c2 = _sc_call()(
        y0, y1, y2, ui_edge_index, s_edge_index, k_edge_index, ones16, zf, zc)
    return _combine(a2d, f0, f1, f2, c0, c1, c2, zb)


# speedup vs baseline: 1.0061x; 1.0061x over previous
import functools

import jax
import jax.numpy as jnp
from jax import lax
from jax.experimental import pallas as pl
from jax.experimental.pallas import tpu as pltpu
from jax.experimental.pallas import tpu_sc as plsc

N = 10000
E = 320000
D = 128
DA = 144
NC, NS = 2, 16
CB = 40
NB = 5
NCH = E // (NC * NS * CB)
HCH = NCH // 2
RPT = N // NS
BN = 2000


def _mm_body(a_ref, x0_ref, x1_ref, x2_ref, wl_ref, wr_ref, bb_ref,
             y0_ref, y1_ref, y2_ref, zb_ref):
    av = a_ref[0, 0]
    weights = (1.0, av, 1.0 - av)
    zb = bb_ref[0][None, :]
    for g, (x_ref, y_ref) in enumerate(
            ((x0_ref, y0_ref), (x1_ref, y1_ref), (x2_ref, y2_ref))):
        x = x_ref[...]
        y_ref[...] = jnp.dot(x, wl_ref[g], preferred_element_type=jnp.float32)
        zb = zb + weights[g] * jnp.dot(
            x, wr_ref[g], preferred_element_type=jnp.float32)
    zb_ref[...] = zb


def _matmuls(a2d, x0, x1, x2, wlts, wrts, bb):
    xspec = pl.BlockSpec((BN, D), lambda i: (i, 0))
    wspec = pl.BlockSpec((3, D, D), lambda i: (0, 0, 0))
    return pl.pallas_call(
        _mm_body,
        grid=(N // BN,),
        in_specs=[pl.BlockSpec(memory_space=pltpu.SMEM),
                  xspec, xspec, xspec, wspec, wspec,
                  pl.BlockSpec((1, D), lambda i: (0, 0))],
        out_specs=[xspec, xspec, xspec, xspec],
        out_shape=[jax.ShapeDtypeStruct((N, D), jnp.float32)] * 4,
    )(a2d, x0, x1, x2, wlts, wrts, bb)


def _sc_body(y0, y1, y2, e0, e1, e2, ones16, zf, zc,
             f0, f1, f2, c0, c1, c2,
             accf, accc, onesb, srcb, dstb, *bufs_and_sems):
    bufs = bufs_and_sems[:NB]
    gsems = bufs_and_sems[NB:2 * NB]
    fsems = bufs_and_sems[2 * NB:3 * NB]
    csems = bufs_and_sems[3 * NB:4 * NB]
    c = lax.axis_index("c")
    s = lax.axis_index("s")
    w = c * NS + s
    r0 = s * RPT
    ebase = w * NCH * CB

    pltpu.sync_copy(ones16, onesb)

    for y_hbm, e_hbm, fo, co in ((y0, e0, f0, c0), (y1, e1, f1, c1),
                                 (y2, e2, f2, c2)):
        pltpu.sync_copy(zf, accf.at[pl.ds(r0, RPT)])
        pltpu.sync_copy(zc, accc.at[pl.ds(r0, RPT)])
        plsc.subcore_barrier()

        for h in range(NCH // HCH):
            hbase = ebase + h * HCH * CB
            pltpu.sync_copy(e_hbm.at[0, pl.ds(hbase, HCH * CB)], srcb)
            pltpu.sync_copy(e_hbm.at[1, pl.ds(hbase, HCH * CB)], dstb)

            def gather(ch, j, y_hbm=y_hbm):
                return pltpu.make_async_copy(
                    y_hbm.at[srcb.at[pl.ds(ch * CB, CB)]], bufs[j], gsems[j])

            for j in range(NB):
                gather(j, j).start()

            def body(i, carry, gather=gather):
                base = NB * i
                scs = []
                for j in range(NB):
                    idx = dstb.at[pl.ds((base + j) * CB, CB)]
                    gather(base + j, j).wait()
                    scs.append(pltpu.async_copy(
                        bufs[j], accf.at[idx], fsems[j], add=True))
                    scs.append(pltpu.async_copy(
                        onesb, accc.at[idx], csems[j], add=True))
                for j in range(NB):
                    scs[2 * j].wait()
                    scs[2 * j + 1].wait()

                    @pl.when(i < HCH // NB - 1)
                    def _(j=j):
                        gather(base + NB + j, j).start()

                return carry

            lax.fori_loop(0, HCH // NB, body, 0)
        plsc.subcore_barrier()
        f8 = s * (RPT - 1)
        pltpu.sync_copy(accf.at[pl.ds(f8, RPT - 1)],
                        fo.at[c, pl.ds(f8, RPT - 1)])

        @pl.when(s == 0)
        def _(fo=fo):
            tail = NS * (RPT - 1)
            pltpu.sync_copy(accf.at[pl.ds(tail, N - tail)],
                            fo.at[c, pl.ds(tail, N - tail)])

        pltpu.sync_copy(accc.at[pl.ds(r0, RPT)], co.at[c, pl.ds(r0, RPT)])


@functools.cache
def _sc_call():
    return pl.kernel(
        _sc_body,
        out_type=[jax.ShapeDtypeStruct((NC, N, D), jnp.float32)] * 3
                 + [jax.ShapeDtypeStruct((NC, N, DA - D), jnp.float32)] * 3,
        mesh=plsc.VectorSubcoreMesh(
            core_axis_name="c", subcore_axis_name="s",
            num_cores=NC, num_subcores=NS),
        scratch_types=[
            pltpu.VMEM_SHARED((N, D), jnp.float32),
            pltpu.VMEM_SHARED((N, DA - D), jnp.float32),
            pltpu.VMEM((CB, DA - D), jnp.float32),
            pltpu.VMEM((HCH * CB,), jnp.int32),
            pltpu.VMEM((HCH * CB,), jnp.int32),
        ] + [pltpu.VMEM((CB, D), jnp.float32)] * NB
          + [pltpu.SemaphoreType.DMA] * (3 * NB),
        compiler_params=pltpu.CompilerParams(use_tc_tiling_on_sc=False),
    )


def _comb_body(a_ref, f0_ref, f1_ref, f2_ref, c0_ref, c1_ref, c2_ref,
               zb_ref, out_ref):
    av = a_ref[0, 0]
    weights = (1.0, av, 1.0 - av)
    tot = zb_ref[...]
    for g, (f_ref, c_ref) in enumerate(
            ((f0_ref, c0_ref), (f1_ref, c1_ref), (f2_ref, c2_ref))):
        sm = f_ref[0] + f_ref[1]
        cnt = c_ref[0, :, 0] + c_ref[1, :, 0]
        tot = tot + weights[g] * (sm / jnp.maximum(cnt, 1.0)[:, None])
    out_ref[...] = tot


def _combine(a2d, f0, f1, f2, c0, c1, c2, zb):
    fspec = pl.BlockSpec((NC, BN, D), lambda i: (0, i, 0))
    cspec = pl.BlockSpec((NC, BN, DA - D), lambda i: (0, i, 0))
    return pl.pallas_call(
        _comb_body,
        grid=(N // BN,),
        in_specs=[
            pl.BlockSpec(memory_space=pltpu.SMEM),
            fspec, fspec, fspec, cspec, cspec, cspec,
            pl.BlockSpec((BN, D), lambda i: (i, 0)),
        ],
        out_specs=pl.BlockSpec((BN, D), lambda i: (i, 0)),
        out_shape=jax.ShapeDtypeStruct((N, D), jnp.float32),
    )(a2d, f0, f1, f2, c0, c1, c2, zb)


def kernel(ui_x, ui_edge_index, s_x, s_edge_index, k_x, k_edge_index, a,
           W_l_ui, b_l_ui, W_r_ui, W_l_s, b_l_s, W_r_s, W_l_k, b_l_k, W_r_k):
    wlts = jnp.stack([W_l_ui.T, W_l_s.T, W_l_k.T])
    wrts = jnp.stack([W_r_ui.T, W_r_s.T, W_r_k.T])
    a2d = jnp.reshape(a, (1, 1))
    bb = (b_l_ui + a * b_l_s + (1.0 - a) * b_l_k).reshape(1, D)
    y0, y1, y2, zb = _matmuls(a2d, ui_x, s_x, k_x, wlts, wrts, bb)
    ones16 = jnp.concatenate(
        [jnp.ones((CB, 1), jnp.float32),
         jnp.zeros((CB, DA - D - 1), jnp.float32)], axis=1)
    zf = jnp.zeros((RPT, D), jnp.float32)
    zc = jnp.zeros((RPT, DA - D), jnp.float32)
    f0, f1, f2, c0, c1, c2 = _sc_call()(
        y0, y1, y2, ui_edge_index, s_edge_index, k_edge_index, ones16, zf, zc)
    return _combine(a2d, f0, f1, f2, c0, c1, c2, zb)
